# baseline (speedup 1.0000x reference)
import functools

import jax
import jax.numpy as jnp
from jax import lax
from jax.experimental import pallas as pl
from jax.experimental.pallas import tpu as pltpu
from jax.experimental.pallas import tpu_sc as plsc

KNN_K = 16
_NC, _NS = 2, 16
_NW = _NC * _NS
_F32 = jnp.float32


def _knn_body(k, n, mult, rpb, rows_ref, cols_ref, idx_ref):
    rows = rows_ref[0]
    cols = cols_ref[0]
    dn = (((1,), (1,)), ((), ()))
    g = lax.dot_general(rows, cols, dn, preferred_element_type=_F32)
    sq_r = jnp.sum(rows * rows, axis=1, keepdims=True)
    ones = jnp.ones((1, 3), _F32)
    sq_c = lax.dot_general(ones, cols * cols, dn,
                           preferred_element_type=_F32)
    d = jnp.maximum(sq_r + sq_c - 2.0 * g, 0.0)
    r = d.shape[0]
    iota = lax.broadcasted_iota(jnp.int32, (r, n), 1)
    e = (lax.bitcast_convert_type(d, jnp.int32) & jnp.int32(~0xFFF)) | iota
    klane = lax.broadcasted_iota(jnp.int32, (r, k), 1)
    big = jnp.int32(2**31 - 1)
    acc = jnp.zeros((r, k), jnp.int32)
    for kk in range(k):
        m = jnp.min(e, axis=1, keepdims=True)
        acc = jnp.where(klane == kk, m & jnp.int32(0xFFF), acc)
        e = jnp.where(e == m, big, e)
    idx_ref[...] = jnp.transpose(acc) * mult + pl.program_id(0) * rpb


@functools.cache
def _knn_call(b, n, r, mult, rpb):
    k = KNN_K
    return pl.pallas_call(
        functools.partial(_knn_body, k, n, mult, rpb),
        grid=(b, n // r),
        in_specs=[
            pl.BlockSpec((1, r, 3), lambda bi, i: (bi, i, 0)),
            pl.BlockSpec((1, n, 3), lambda bi, i: (bi, 0, 0)),
        ],
        out_specs=pl.BlockSpec((k, r), lambda bi, i: (0, bi * (n // r) + i)),
        out_shape=jax.ShapeDtypeStruct((k, b * n), jnp.int32),
    )


def _gather_body(b_per_w, ch, dp, table_hbm, idx_hbm, out_hbm, idx_v, rows_v, sem):
    wid = lax.axis_index("s") * _NC + lax.axis_index("c")
    base = wid * b_per_w
    for it in range(b_per_w // ch):
        off = base + it * ch
        pltpu.sync_copy(idx_hbm.at[pl.ds(off, ch)], idx_v)
        descs = []
        for c in range(ch // 128):
            descs.append(pltpu.async_copy(
                table_hbm.at[idx_v.at[pl.ds(c * 128, 128)]],
                rows_v.at[pl.ds(c * 128, 128)], sem))
        for dsc in descs:
            dsc.wait()
        pltpu.sync_copy(rows_v, out_hbm.at[pl.ds(off, ch)])


@functools.cache
def _gather_call(rows_t, dp, m):
    b_per_w = m // _NW
    ch = min(b_per_w, 1024)
    mesh = plsc.VectorSubcoreMesh(core_axis_name="c", subcore_axis_name="s")
    return pl.kernel(
        functools.partial(_gather_body, b_per_w, ch, dp),
        out_type=jax.ShapeDtypeStruct((m, dp), _F32),
        mesh=mesh,
        compiler_params=pltpu.CompilerParams(use_tc_tiling_on_sc=False),
        scratch_types=[
            pltpu.VMEM((ch,), jnp.int32),
            pltpu.VMEM((ch, dp), _F32),
            pltpu.SemaphoreType.DMA,
        ],
    )


def _mm(a, b):
    return jnp.dot(a, b, preferred_element_type=_F32)


def _lfa_body(k, co, head, *refs):
    if head == 'enc':
        (g_ref, ctr_ref, wc, wg1, wd, bnb, wg2, wm2, bm, wa,
         wh, bh, noise_ref, out_ref) = refs
    elif head == 'up':
        (g_ref, ctr_ref, wc, wg1, wd, bnb, wg2, wm2, bm, wa,
         wh, bh, ta_ref, tb_ref) = refs
    else:
        (g_ref, ctr_ref, wc, wg1, wd, bnb, wg2, wm2, bm, wa, out_ref) = refs
    ctr = ctr_ref[...]
    Wc, Wg1, Wd, Bnb = wc[...], wg1[...], wd[...], bnb[...]
    Wg2, Wm2, Bm, Wa = wg2[...], wm2[...], bm[...], wa[...]
    r = ctr.shape[0]
    z13 = jnp.zeros((r, 13), _F32)
    gall = jnp.concatenate([g_ref[kk] for kk in range(k)], axis=0)
    ctrk = jnp.concatenate([ctr] * k, axis=0)
    rel = ctrk - gall[:, 0:3]
    cn = Wc.shape[1]
    d2 = _mm(rel * rel, jnp.ones((3, cn), _F32))
    nf = jnp.maximum(_mm(ctrk, Wc) + _mm(gall, Wg1)
                     + jnp.sqrt(d2) * Wd + Bnb, 0.0)
    x = jnp.maximum(_mm(gall, Wg2) + _mm(nf, Wm2) + Bm, 0.0)
    a = _mm(x, Wa)
    m = a[0:r]
    for kk in range(1, k):
        m = jnp.maximum(m, a[kk * r:(kk + 1) * r])
    em = jnp.exp(a - jnp.concatenate([m] * k, axis=0))
    p = em * x
    s, o = em[0:r], p[0:r]
    for kk in range(1, k):
        s = s + em[kk * r:(kk + 1) * r]
        o = o + p[kk * r:(kk + 1) * r]
    out = o / s
    if head == 'enc':
        f16 = _mm(out, wh[...]) + bh[...] + noise_ref[...]
        out_ref[...] = jnp.concatenate([ctr, z13, f16], axis=1)
    elif head == 'up':
        h = co // 2
        Wh, Bh = wh[...], bh[...]
        ca = ctr + _mm(out[:, 0:h], Wh) + Bh
        cb = ctr + _mm(out[:, h:co], Wh) + Bh
        ta_ref[...] = jnp.concatenate([ca, z13, out[:, 0:h]], axis=1)
        tb_ref[...] = jnp.concatenate([cb, z13, out[:, h:co]], axis=1)
    else:
        out_ref[...] = jnp.concatenate([ctr, z13, out], axis=1)


@functools.cache
def _lfa_call(mpts, cn, co, dp, head, r=1024):
    k = KNN_K

    def full(shape):
        return pl.BlockSpec(shape, lambda i: tuple(0 for _ in shape))

    in_specs = [
        pl.BlockSpec((k, r, dp), lambda i: (0, i, 0)),
        pl.BlockSpec((r, 3), lambda i: (i, 0)),
        full((3, cn)), full((dp, cn)), full((1, cn)), full((1, cn)),
        full((dp, co)), full((cn, co)), full((1, co)), full((co, co)),
    ]
    if head == 'enc':
        dpn = 32
        in_specs += [full((co, 16)), full((1, 16)),
                     pl.BlockSpec((r, 16), lambda i: (i, 0))]
        out_specs = pl.BlockSpec((r, dpn), lambda i: (i, 0))
        out_shape = jax.ShapeDtypeStruct((mpts, dpn), _F32)
    elif head == 'up':
        dpn = 16 + co // 2
        in_specs += [full((co // 2, 3)), full((1, 3))]
        out_specs = [pl.BlockSpec((r, dpn), lambda i: (i, 0)),
                     pl.BlockSpec((r, dpn), lambda i: (i, 0))]
        out_shape = [jax.ShapeDtypeStruct((mpts, dpn), _F32),
                     jax.ShapeDtypeStruct((mpts, dpn), _F32)]
    else:
        dpn = 16 + co
        out_specs = pl.BlockSpec((r, dpn), lambda i: (i, 0))
        out_shape = jax.ShapeDtypeStruct((mpts, dpn), _F32)
    return pl.pallas_call(
        functools.partial(_lfa_body, k, co, head),
        grid=(mpts // r,),
        in_specs=in_specs,
        out_specs=out_specs,
        out_shape=out_shape,
    )


def _prep_weights(lp, ci, dp):
    wnb, wm = lp['Wnb'], lp['Wm']
    cn, co = wnb.shape[1], wm.shape[1]
    wc = wnb[0:3] + wnb[6:9]
    wg1 = jnp.zeros((dp, cn), _F32).at[0:3].set(wnb[3:6] - wnb[6:9])
    wg2 = jnp.zeros((dp, co), _F32).at[16:16 + ci].set(wm[0:ci])
    return (wc, wg1, wnb[9:10], lp['bnb'][None, :],
            wg2, wm[ci:], lp['bm'][None, :], lp['Wa'])


def _lfa_layer(cx, table, flat_idx, lp, ci, head=None, extra=()):
    b, n, _ = cx.shape
    cn, co = lp['Wnb'].shape[1], lp['Wm'].shape[1]
    dp = table.shape[-1]
    mpts = b * n
    ctr = cx.reshape(mpts, 3)
    g = _gather_call(table.shape[0], dp, KNN_K * mpts)(table, flat_idx)
    g3 = g.reshape(KNN_K, mpts, dp)
    w = _prep_weights(lp, ci, dp)
    return _lfa_call(mpts, cn, co, dp, head)(g3, ctr, *w, *extra)


def kernel(xyz, params):
    p = params
    b, n, _ = xyz.shape
    cx = xyz.astype(_F32)
    x2 = cx.reshape(b * n, 3)
    z13 = jnp.zeros((b * n, 13), _F32)
    tbl = jnp.concatenate([x2, z13, x2, z13], axis=1)
    fi = _knn_call(b, n, 256, 1, n)(cx, cx).reshape(-1)
    tbl = _lfa_layer(cx, tbl, fi, p['l0'], 3)
    tbl = _lfa_layer(cx, tbl, fi, p['l1'], 32)
    cx, n = cx[:, ::2], n // 2
    fi = _knn_call(b, n, 256, 2, 2 * n)(cx, cx).reshape(-1)
    tbl = _lfa_layer(cx, tbl, fi, p['l2'], 32)
    tbl = _lfa_layer(cx, tbl, fi >> 1, p['l3'], 64)
    cx, n = cx[:, ::2], n // 2
    fi = _knn_call(b, n, 256, 2, 2 * n)(cx, cx).reshape(-1)
    tbl = _lfa_layer(cx, tbl, fi, p['l4'], 64)
    with jax.ensure_compile_time_eval():
        noise = jax.random.uniform(jax.random.key(7), (b * n, 16), _F32,
                                   -0.5, 0.5)
    tbl = _lfa_layer(cx, tbl, fi >> 1, p['l5'], 64, head='enc',
                     extra=(p['Wout'], p['bout'][None, :], noise))
    ta, tb = _lfa_layer(cx, tbl, fi >> 1, p['l6'], 16, head='up',
                        extra=(p['Wp0'], p['bp0'][None, :]))
    tbl = jnp.concatenate([ta, tb], axis=0)
    cx = jnp.stack([ta[:, 0:3].reshape(b, n, 3), tb[:, 0:3].reshape(b, n, 3)],
                   axis=2).reshape(b, 2 * n, 3)
    pp, n = n, 2 * n
    q = _knn_call(b, n, 256, 1, 0)(cx, cx).reshape(KNN_K, b, n)
    bb = jnp.arange(b, dtype=jnp.int32)[None, :, None]
    fi = ((q & 1) * (b * pp) + bb * pp + (q >> 1)).reshape(-1)
    ta, tb = _lfa_layer(cx, tbl, fi, p['l7'], 32, head='up',
                        extra=(p['Wp1'], p['bp1'][None, :]))
    return jnp.stack([ta[:, 0:3].reshape(b, n, 3), tb[:, 0:3].reshape(b, n, 3)],
                     axis=2).reshape(b, 2 * n, 3)


# speedup vs baseline: 1.0033x; 1.0033x over previous
import functools

import jax
import jax.numpy as jnp
from jax import lax
from jax.experimental import pallas as pl
from jax.experimental.pallas import tpu as pltpu
from jax.experimental.pallas import tpu_sc as plsc

KNN_K = 16
_NC, _NS = 2, 16
_NW = _NC * _NS
_F32 = jnp.float32


def _knn_body(k, n, mult, rpb, rows_ref, cols_ref, idx_ref):
    rows = rows_ref[0]
    cols = cols_ref[0]
    dn = (((1,), (1,)), ((), ()))
    g = lax.dot_general(rows, cols, dn, preferred_element_type=_F32)
    sq_r = jnp.sum(rows * rows, axis=1, keepdims=True)
    ones = jnp.ones((1, 3), _F32)
    sq_c = lax.dot_general(ones, cols * cols, dn,
                           preferred_element_type=_F32)
    d = jnp.maximum(sq_r + sq_c - 2.0 * g, 0.0)
    r = d.shape[0]
    iota = lax.broadcasted_iota(jnp.int32, (r, n), 1)
    e = (lax.bitcast_convert_type(d, jnp.int32) & jnp.int32(~0xFFF)) | iota
    klane = lax.broadcasted_iota(jnp.int32, (r, k), 1)
    big = jnp.int32(2**31 - 1)
    acc = jnp.zeros((r, k), jnp.int32)
    for kk in range(k):
        m = jnp.min(e, axis=1, keepdims=True)
        acc = jnp.where(klane == kk, m & jnp.int32(0xFFF), acc)
        e = jnp.where(e == m, big, e)
    idx_ref[...] = jnp.transpose(acc) * mult + pl.program_id(0) * rpb


@functools.cache
def _knn_call(b, n, r, mult, rpb):
    k = KNN_K
    return pl.pallas_call(
        functools.partial(_knn_body, k, n, mult, rpb),
        grid=(b, n // r),
        in_specs=[
            pl.BlockSpec((1, r, 3), lambda bi, i: (bi, i, 0)),
            pl.BlockSpec((1, n, 3), lambda bi, i: (bi, 0, 0)),
        ],
        out_specs=pl.BlockSpec((k, r), lambda bi, i: (0, bi * (n // r) + i)),
        out_shape=jax.ShapeDtypeStruct((k, b * n), jnp.int32),
    )


def _gather_body(b_per_w, ch, dp, table_hbm, idx_hbm, out_hbm, idx_v, rows_v, sem):
    wid = lax.axis_index("s") * _NC + lax.axis_index("c")
    base = wid * b_per_w
    for it in range(b_per_w // ch):
        off = base + it * ch
        pltpu.sync_copy(idx_hbm.at[pl.ds(off, ch)], idx_v)
        descs = []
        for c in range(ch // 128):
            descs.append(pltpu.async_copy(
                table_hbm.at[idx_v.at[pl.ds(c * 128, 128)]],
                rows_v.at[pl.ds(c * 128, 128)], sem))
        for dsc in descs:
            dsc.wait()
        pltpu.sync_copy(rows_v, out_hbm.at[pl.ds(off, ch)])


@functools.cache
def _gather_call(rows_t, dp, m):
    b_per_w = m // _NW
    ch = min(b_per_w, 1024)
    mesh = plsc.VectorSubcoreMesh(core_axis_name="c", subcore_axis_name="s")
    return pl.kernel(
        functools.partial(_gather_body, b_per_w, ch, dp),
        out_type=jax.ShapeDtypeStruct((m, dp), _F32),
        mesh=mesh,
        compiler_params=pltpu.CompilerParams(use_tc_tiling_on_sc=False),
        scratch_types=[
            pltpu.VMEM((ch,), jnp.int32),
            pltpu.VMEM((ch, dp), _F32),
            pltpu.SemaphoreType.DMA,
        ],
    )


def _mm(a, b):
    return jnp.dot(a, b, preferred_element_type=_F32)


def _lfa_body(k, co, head, *refs):
    if head == 'enc':
        (g_ref, ctr_ref, wc, wg1, wd, bnb, wg2, wm2, bm, wa,
         wh, bh, noise_ref, out_ref) = refs
    elif head == 'up':
        (g_ref, ctr_ref, wc, wg1, wd, bnb, wg2, wm2, bm, wa,
         wh, bh, ta_ref, tb_ref) = refs
    else:
        (g_ref, ctr_ref, wc, wg1, wd, bnb, wg2, wm2, bm, wa, out_ref) = refs
    ctr = ctr_ref[...]
    Wc, Wg1, Wd, Bnb = wc[...], wg1[...], wd[...], bnb[...]
    Wg2, Wm2, Bm, Wa = wg2[...], wm2[...], bm[...], wa[...]
    r = ctr.shape[0]
    z13 = jnp.zeros((r, 13), _F32)
    gall = jnp.concatenate([g_ref[kk] for kk in range(k)], axis=0)
    ctrk = jnp.concatenate([ctr] * k, axis=0)
    rel = ctrk - gall[:, 0:3]
    cn = Wc.shape[1]
    d2 = _mm(rel * rel, jnp.ones((3, cn), _F32))
    nf = jnp.maximum(_mm(ctrk, Wc) + _mm(gall, Wg1)
                     + jnp.sqrt(d2) * Wd + Bnb, 0.0)
    x = jnp.maximum(_mm(gall, Wg2) + _mm(nf, Wm2) + Bm, 0.0)
    a = _mm(x, Wa)
    m = a[0:r]
    for kk in range(1, k):
        m = jnp.maximum(m, a[kk * r:(kk + 1) * r])
    em = jnp.exp(a - jnp.concatenate([m] * k, axis=0))
    p = em * x
    s, o = em[0:r], p[0:r]
    for kk in range(1, k):
        s = s + em[kk * r:(kk + 1) * r]
        o = o + p[kk * r:(kk + 1) * r]
    out = o / s
    if head == 'enc':
        f16 = _mm(out, wh[...]) + bh[...] + noise_ref[...]
        out_ref[...] = jnp.concatenate([ctr, z13, f16], axis=1)
    elif head == 'up':
        h = co // 2
        Wh, Bh = wh[...], bh[...]
        ca = ctr + _mm(out[:, 0:h], Wh) + Bh
        cb = ctr + _mm(out[:, h:co], Wh) + Bh
        ta_ref[...] = jnp.concatenate([ca, z13, out[:, 0:h]], axis=1)
        tb_ref[...] = jnp.concatenate([cb, z13, out[:, h:co]], axis=1)
    else:
        out_ref[...] = jnp.concatenate([ctr, z13, out], axis=1)


@functools.cache
def _lfa_call(mpts, cn, co, dp, head, r=512):
    k = KNN_K

    def full(shape):
        return pl.BlockSpec(shape, lambda i: tuple(0 for _ in shape))

    in_specs = [
        pl.BlockSpec((k, r, dp), lambda i: (0, i, 0)),
        pl.BlockSpec((r, 3), lambda i: (i, 0)),
        full((3, cn)), full((dp, cn)), full((1, cn)), full((1, cn)),
        full((dp, co)), full((cn, co)), full((1, co)), full((co, co)),
    ]
    if head == 'enc':
        dpn = 32
        in_specs += [full((co, 16)), full((1, 16)),
                     pl.BlockSpec((r, 16), lambda i: (i, 0))]
        out_specs = pl.BlockSpec((r, dpn), lambda i: (i, 0))
        out_shape = jax.ShapeDtypeStruct((mpts, dpn), _F32)
    elif head == 'up':
        dpn = 16 + co // 2
        in_specs += [full((co // 2, 3)), full((1, 3))]
        out_specs = [pl.BlockSpec((r, dpn), lambda i: (i, 0)),
                     pl.BlockSpec((r, dpn), lambda i: (i, 0))]
        out_shape = [jax.ShapeDtypeStruct((mpts, dpn), _F32),
                     jax.ShapeDtypeStruct((mpts, dpn), _F32)]
    else:
        dpn = 16 + co
        out_specs = pl.BlockSpec((r, dpn), lambda i: (i, 0))
        out_shape = jax.ShapeDtypeStruct((mpts, dpn), _F32)
    return pl.pallas_call(
        functools.partial(_lfa_body, k, co, head),
        grid=(mpts // r,),
        in_specs=in_specs,
        out_specs=out_specs,
        out_shape=out_shape,
    )


def _prep_weights(lp, ci, dp):
    wnb, wm = lp['Wnb'], lp['Wm']
    cn, co = wnb.shape[1], wm.shape[1]
    wc = wnb[0:3] + wnb[6:9]
    wg1 = jnp.zeros((dp, cn), _F32).at[0:3].set(wnb[3:6] - wnb[6:9])
    wg2 = jnp.zeros((dp, co), _F32).at[16:16 + ci].set(wm[0:ci])
    return (wc, wg1, wnb[9:10], lp['bnb'][None, :],
            wg2, wm[ci:], lp['bm'][None, :], lp['Wa'])


def _lfa_layer(cx, table, flat_idx, lp, ci, head=None, extra=()):
    b, n, _ = cx.shape
    cn, co = lp['Wnb'].shape[1], lp['Wm'].shape[1]
    dp = table.shape[-1]
    mpts = b * n
    ctr = cx.reshape(mpts, 3)
    g = _gather_call(table.shape[0], dp, KNN_K * mpts)(table, flat_idx)
    g3 = g.reshape(KNN_K, mpts, dp)
    w = _prep_weights(lp, ci, dp)
    return _lfa_call(mpts, cn, co, dp, head)(g3, ctr, *w, *extra)


def kernel(xyz, params):
    p = params
    b, n, _ = xyz.shape
    cx = xyz.astype(_F32)
    x2 = cx.reshape(b * n, 3)
    z13 = jnp.zeros((b * n, 13), _F32)
    tbl = jnp.concatenate([x2, z13, x2, z13], axis=1)
    fi = _knn_call(b, n, 256, 1, n)(cx, cx).reshape(-1)
    tbl = _lfa_layer(cx, tbl, fi, p['l0'], 3)
    tbl = _lfa_layer(cx, tbl, fi, p['l1'], 32)
    cx, n = cx[:, ::2], n // 2
    fi = _knn_call(b, n, 256, 2, 2 * n)(cx, cx).reshape(-1)
    tbl = _lfa_layer(cx, tbl, fi, p['l2'], 32)
    tbl = _lfa_layer(cx, tbl, fi >> 1, p['l3'], 64)
    cx, n = cx[:, ::2], n // 2
    fi = _knn_call(b, n, 256, 2, 2 * n)(cx, cx).reshape(-1)
    tbl = _lfa_layer(cx, tbl, fi, p['l4'], 64)
    with jax.ensure_compile_time_eval():
        noise = jax.random.uniform(jax.random.key(7), (b * n, 16), _F32,
                                   -0.5, 0.5)
    tbl = _lfa_layer(cx, tbl, fi >> 1, p['l5'], 64, head='enc',
                     extra=(p['Wout'], p['bout'][None, :], noise))
    ta, tb = _lfa_layer(cx, tbl, fi >> 1, p['l6'], 16, head='up',
                        extra=(p['Wp0'], p['bp0'][None, :]))
    tbl = jnp.concatenate([ta, tb], axis=0)
    cx = jnp.stack([ta[:, 0:3].reshape(b, n, 3), tb[:, 0:3].reshape(b, n, 3)],
                   axis=2).reshape(b, 2 * n, 3)
    pp, n = n, 2 * n
    q = _knn_call(b, n, 256, 1, 0)(cx, cx).reshape(KNN_K, b, n)
    bb = jnp.arange(b, dtype=jnp.int32)[None, :, None]
    fi = ((q & 1) * (b * pp) + bb * pp + (q >> 1)).reshape(-1)
    ta, tb = _lfa_layer(cx, tbl, fi, p['l7'], 32, head='up',
                        extra=(p['Wp1'], p['bp1'][None, :]))
    return jnp.stack([ta[:, 0:3].reshape(b, n, 3), tb[:, 0:3].reshape(b, n, 3)],
                     axis=2).reshape(b, 2 * n, 3)
